# Initial kernel scaffold; baseline (speedup 1.0000x reference)
import jax
import jax.numpy as jnp
from jax.experimental import pallas as pl


def kernel(inputs, target, params):
    raise NotImplementedError("write your pallas kernel here")


# speedup vs baseline: 3.2012x; 3.2012x over previous
import functools

import jax
import jax.numpy as jnp
from jax import lax
from jax.experimental import pallas as pl
from jax.experimental.pallas import tpu as pltpu
from jax.experimental.pallas import tpu_sc as plsc

WORD_COUNT = 50000
EMB = 128
HID = 256
MAX_OOVS = 100
B = 32
T = 128
TL = 16
NSTEP = TL - 1
VPAD = 50176
VTILE = 1024
NVT = VPAD // VTILE

_NC, _NS = 2, 16
NW = _NC * _NS


def _sc_gather(table, idx_enc, idx_dec):
    enc_per_w = idx_enc.shape[0] // NW
    dec_per_w = idx_dec.shape[0] // NW
    mesh = plsc.VectorSubcoreMesh(
        core_axis_name="c", subcore_axis_name="s",
        num_cores=_NC, num_subcores=_NS)

    @functools.partial(
        pl.kernel,
        mesh=mesh,
        out_type=[
            jax.ShapeDtypeStruct((idx_enc.shape[0], EMB), jnp.float32),
            jax.ShapeDtypeStruct((idx_dec.shape[0], EMB), jnp.float32),
        ],
        scratch_types=[
            pltpu.VMEM((enc_per_w,), jnp.int32),
            pltpu.VMEM((enc_per_w, EMB), jnp.float32),
            pltpu.VMEM((dec_per_w,), jnp.int32),
            pltpu.VMEM((dec_per_w, EMB), jnp.float32),
            pltpu.SemaphoreType.DMA,
        ],
    )
    def k(table_hbm, ie_hbm, id_hbm, oe_hbm, od_hbm, ie_v, re_v, id_v, rd_v, sem):
        wid = lax.axis_index("s") * _NC + lax.axis_index("c")
        be = wid * enc_per_w
        bd = wid * dec_per_w
        pltpu.sync_copy(ie_hbm.at[pl.ds(be, enc_per_w)], ie_v)
        pltpu.async_copy(table_hbm.at[ie_v], re_v, sem).wait()
        pltpu.sync_copy(re_v, oe_hbm.at[pl.ds(be, enc_per_w)])
        pltpu.sync_copy(id_hbm.at[pl.ds(bd, dec_per_w)], id_v)
        pltpu.async_copy(table_hbm.at[id_v], rd_v, sem).wait()
        pltpu.sync_copy(rd_v, od_hbm.at[pl.ds(bd, dec_per_w)])

    return k(table, idx_enc, idx_dec)


def _tc_core_body(
    xs_ref, de_ref, scat_ref,
    wihf_ref, whhf_ref, bf_ref, wihb_ref, whhb_ref, bb_ref,
    wihd_ref, whhd_ref, bd_ref,
    u1_ref, u2_ref, q_ref,
    whw_ref, wsw_ref, wxw_ref, pgb_ref,
    v1w_ref, v1b_ref,
    hidden_ref, vals_ref, idx2_ref, pg_ref, cov_ref,
    hsf, hsb, hall,
):
    def cell(x, h, c, wih, whh, bias):
        gg = jnp.dot(x, wih, preferred_element_type=jnp.float32)
        gg = gg + jnp.dot(h, whh, preferred_element_type=jnp.float32) + bias
        ii = jax.nn.sigmoid(gg[:, 0:HID])
        ff = jax.nn.sigmoid(gg[:, HID:2 * HID])
        cc = jnp.tanh(gg[:, 2 * HID:3 * HID])
        oo = jax.nn.sigmoid(gg[:, 3 * HID:4 * HID])
        c2 = ff * c + ii * cc
        return oo * jnp.tanh(c2), c2

    wihf = wihf_ref[...]
    whhf = whhf_ref[...]
    bfv = bf_ref[...]
    wihb = wihb_ref[...]
    whhb = whhb_ref[...]
    bbv = bb_ref[...]

    def enc_step(t, carry):
        hf, cf, hb, cb = carry
        xf = xs_ref[t]
        xb = xs_ref[T - 1 - t]
        hf2, cf2 = cell(xf, hf, cf, wihf, whhf, bfv)
        hb2, cb2 = cell(xb, hb, cb, wihb, whhb, bbv)
        hsf[t] = hf2
        hsb[T - 1 - t] = hb2
        return hf2, cf2, hb2, cb2

    z = jnp.zeros((B, HID), jnp.float32)
    lax.fori_loop(0, T, enc_step, (z, z, z, z), unroll=2)

    wihd = wihd_ref[...]
    whhd = whhd_ref[...]
    bdv = bd_ref[...]

    def dec_step(i, carry):
        hd, cd = carry
        hd2, cd2 = cell(de_ref[i], hd, cd, wihd, whhd, bdv)
        hall[i] = hd2
        return hd2, cd2

    lax.fori_loop(0, NSTEP, dec_step, (z, z), unroll=4)

    hsf_all = hsf[...]
    hsb_all = hsb[...]
    hall_all = hall[...]

    u1 = u1_ref[...].reshape(1, 1, HID)
    u2 = u2_ref[...].reshape(1, 1, HID)
    qv = q_ref[...].reshape(1, 1, HID)

    E_tb = jnp.sum(hsf_all * u1, axis=-1) + jnp.sum(hsb_all * u2, axis=-1)
    w_tb = jnp.exp(E_tb - jnp.max(E_tb, axis=0, keepdims=True))

    s_ib = jnp.sum(hall_all * qv, axis=-1)
    g = jnp.exp(s_ib - jnp.max(s_ib, axis=1, keepdims=True))

    wv = w_tb.reshape(4, 32, B)
    W4 = jnp.sum(wv, axis=0)
    Z = jnp.dot(g, W4, preferred_element_type=jnp.float32)

    hfv = hsf_all.reshape(4, 32, B, HID)
    hbv = hsb_all.reshape(4, 32, B, HID)
    Pf = jnp.sum(wv[..., None] * hfv, axis=0)
    Pb = jnp.sum(wv[..., None] * hbv, axis=0)
    gb = g[:, :, None, None]
    ctx_f = jnp.sum(gb * Pf[None], axis=1) / Z[:, :, None]
    ctx_b = jnp.sum(gb * Pb[None], axis=1) / Z[:, :, None]

    g_exp = jnp.concatenate([g, g, g, g], axis=1)
    w_bt = w_tb.T
    attn = w_bt[None] * g_exp[:, None, :] / Z[:, :, None]

    cov = jnp.zeros((B, T), jnp.float32)
    cl = jnp.zeros((), jnp.float32)
    for i in range(NSTEP):
        cl = cl + jnp.sum(jnp.minimum(attn[i], cov))
        cov = cov + attn[i]
    cov_ref[...] = cl.reshape(1, 1)

    demb = de_ref[0:NSTEP]
    whw = whw_ref[...]
    pg_lin = (
        jnp.sum(ctx_f * whw[:, 0:HID].reshape(1, 1, HID), axis=-1)
        + jnp.sum(ctx_b * whw[:, HID:2 * HID].reshape(1, 1, HID), axis=-1)
        + jnp.sum(hall_all * wsw_ref[...].reshape(1, 1, HID), axis=-1)
        + jnp.sum(demb * wxw_ref[...].reshape(1, 1, EMB), axis=-1)
        + pgb_ref[0, 0]
    )
    pg = jax.nn.sigmoid(pg_lin)
    pg_ref[...] = pg
    valsv = (1.0 - pg)[:, :, None] * attn

    sidx = scat_ref[...]
    eqf = (sidx[:, :, None] == sidx[:, None, :]).astype(jnp.float32)
    t_i = lax.broadcasted_iota(jnp.int32, (T, T), 0)
    t_j = lax.broadcasted_iota(jnp.int32, (T, T), 1)
    tri = (t_j < t_i).astype(jnp.float32)
    first = jnp.sum(eqf * tri[None], axis=2) == 0.0
    firstf = first.astype(jnp.float32)
    v2 = [jnp.sum(eqf * valsv[i][:, None, :], axis=2) for i in range(NSTEP)]
    vals_ref[...] = jnp.stack(v2) * firstf[None]
    idx2_ref[...] = jnp.where(first, sidx, VPAD - 1)

    cat = jnp.concatenate([hall_all, ctx_f, ctx_b], axis=-1).reshape(NSTEP * B, 3 * HID)
    hidden_ref[...] = (
        jnp.dot(cat, v1w_ref[...], preferred_element_type=jnp.float32) + v1b_ref[...]
    )


def _tc_core(xs, de, scat_idx, wihf, whhf, bf, wihb, whhb, bb, wihd, whhd, bd,
             u1, u2, q, whw, wsw, wxw, pgb, v1w, v1b):
    return pl.pallas_call(
        _tc_core_body,
        out_shape=[
            jax.ShapeDtypeStruct((NSTEP * B, 3 * HID), jnp.float32),
            jax.ShapeDtypeStruct((NSTEP, B, T), jnp.float32),
            jax.ShapeDtypeStruct((B, T), jnp.int32),
            jax.ShapeDtypeStruct((NSTEP, B), jnp.float32),
            jax.ShapeDtypeStruct((1, 1), jnp.float32),
        ],
        scratch_shapes=[
            pltpu.VMEM((T, B, HID), jnp.float32),
            pltpu.VMEM((T, B, HID), jnp.float32),
            pltpu.VMEM((NSTEP, B, HID), jnp.float32),
        ],
    )(xs, de, scat_idx, wihf, whhf, bf, wihb, whhb, bb, wihd, whhd, bd,
      u1, u2, q, whw, wsw, wxw, pgb, v1w, v1b)


def _tc_logits_body(h_ref, v2_ref, b2_ref, out_ref):
    out_ref[...] = (
        jnp.dot(h_ref[...], v2_ref[...], preferred_element_type=jnp.float32)
        + b2_ref[...]
    )


def _tc_logits(hidden, v2t_pad, b2_pad):
    return pl.pallas_call(
        _tc_logits_body,
        grid=(NVT,),
        in_specs=[
            pl.BlockSpec((NSTEP * B, 3 * HID), lambda j: (0, 0)),
            pl.BlockSpec((3 * HID, VTILE), lambda j: (0, j)),
            pl.BlockSpec((1, VTILE), lambda j: (0, j)),
        ],
        out_specs=pl.BlockSpec((NSTEP * B, VTILE), lambda j: (0, j)),
        out_shape=jax.ShapeDtypeStruct((NSTEP * B, VPAD), jnp.float32),
    )(hidden, v2t_pad, b2_pad)


def _tc_pev_body(l_ref, pg_ref, out_ref):
    i = pl.program_id(0)
    l = l_ref[...]
    m = jnp.max(l, axis=1, keepdims=True)
    e = jnp.exp(l - m)
    pv = e / jnp.sum(e, axis=1, keepdims=True)
    pgi = jnp.transpose(pg_ref[pl.ds(i, 1), :])
    out_ref[...] = 1.0 / WORD_COUNT / 100.0 + pgi * pv


def _tc_pev(logits, pg):
    return pl.pallas_call(
        _tc_pev_body,
        grid=(NSTEP,),
        in_specs=[
            pl.BlockSpec((B, VPAD), lambda i: (i, 0)),
            pl.BlockSpec((NSTEP, B), lambda i: (0, 0)),
        ],
        out_specs=pl.BlockSpec((B, VPAD), lambda i: (0, i)),
        out_shape=jax.ShapeDtypeStruct((B, NSTEP * VPAD), jnp.float32),
    )(logits, pg)


def _sc_scatter(pev, vals, scat_idx):
    mesh = plsc.VectorSubcoreMesh(
        core_axis_name="c", subcore_axis_name="s",
        num_cores=_NC, num_subcores=_NS)

    @functools.partial(
        pl.kernel,
        mesh=mesh,
        out_type=jax.ShapeDtypeStruct((B * NSTEP * VPAD,), jnp.float32),
        scratch_types=[
            pltpu.VMEM((VPAD,), jnp.float32),
            pltpu.VMEM((T,), jnp.int32),
            pltpu.VMEM((T,), jnp.float32),
        ],
        compiler_params=pltpu.CompilerParams(needs_layout_passes=False),
    )
    def k(pev_hbm, vals_hbm, idx_hbm, out_hbm, row_v, idx_v, val_v):
        b = lax.axis_index("s") * _NC + lax.axis_index("c")
        pltpu.sync_copy(idx_hbm.at[pl.ds(b * T, T)], idx_v)
        for i in range(NSTEP):
            r = b * NSTEP + i
            pltpu.sync_copy(pev_hbm.at[pl.ds(r * VPAD, VPAD)], row_v)
            pltpu.sync_copy(vals_hbm.at[pl.ds((i * B + b) * T, T)], val_v)
            for gk in range(T // 16):
                iv = idx_v[pl.ds(gk * 16, 16)]
                vv = val_v[pl.ds(gk * 16, 16)]
                plsc.addupdate_scatter(row_v, [iv], vv)
            pltpu.sync_copy(row_v, out_hbm.at[pl.ds(r * VPAD, VPAD)])

    return k(pev.reshape(-1), vals.reshape(-1), scat_idx.reshape(-1))


def kernel(inputs, target, params):
    p = params
    unked = jnp.where(inputs >= WORD_COUNT, 1, inputs)
    idx_enc = unked.T.reshape(-1)
    idx_dec = jnp.where(target >= WORD_COUNT, 1, target).T.reshape(-1)
    scat_idx = inputs.reshape(-1).reshape(T, B).T

    enc_emb, dec_emb = _sc_gather(p["embed"], idx_enc, idx_dec)
    xs = enc_emb.reshape(T, B, EMB)
    de = dec_emb.reshape(TL, B, EMB)

    v_w = p["v_w"][0]
    u = p["Wh_w"].T @ v_w
    q = (p["Ws_w"].T @ v_w).reshape(1, HID)
    u1 = u[:HID].reshape(1, HID)
    u2 = u[HID:].reshape(1, HID)
    pg_bias = (p["wh_b"] + p["ws_b"] + p["wx_b"]).reshape(1, 1)

    hidden, vals, idx2, pg, cov = _tc_core(
        xs, de, scat_idx,
        p["enc_Wih_f"].T, p["enc_Whh_f"].T, (p["enc_bih_f"] + p["enc_bhh_f"]).reshape(1, -1),
        p["enc_Wih_b"].T, p["enc_Whh_b"].T, (p["enc_bih_b"] + p["enc_bhh_b"]).reshape(1, -1),
        p["dec_Wih"].T, p["dec_Whh"].T, (p["dec_bih"] + p["dec_bhh"]).reshape(1, -1),
        u1, u2, q,
        p["wh_w"], p["ws_w"].reshape(1, HID), p["wx_w"].reshape(1, EMB), pg_bias,
        p["V1_w"].T, p["V1_b"].reshape(1, -1),
    )

    v2t = jnp.pad(p["V2_w"].T, ((0, 0), (0, VPAD - WORD_COUNT)))
    b2 = jnp.pad(p["V2_b"], (0, VPAD - WORD_COUNT),
                 constant_values=-1e30).reshape(1, VPAD)

    logits = _tc_logits(hidden, v2t, b2)
    pev = _tc_pev(logits, pg)
    pev = _sc_scatter(pev, vals, idx2).reshape(B, NSTEP, VPAD)

    return pev[:, :, :WORD_COUNT + MAX_OOVS], cov.reshape(())


# speedup vs baseline: 3.8523x; 1.2034x over previous
import functools

import jax
import jax.numpy as jnp
from jax import lax
from jax.experimental import pallas as pl
from jax.experimental.pallas import tpu as pltpu
from jax.experimental.pallas import tpu_sc as plsc

WORD_COUNT = 50000
EMB = 128
HID = 256
MAX_OOVS = 100
B = 32
T = 128
TL = 16
NSTEP = TL - 1
VPAD = 50176
VTILE = 1024
NVT = VPAD // VTILE
VOUT = WORD_COUNT + MAX_OOVS
VHEAD = 50048

_NC, _NS = 2, 16
NW = _NC * _NS


def _sc_gather(table, idx_enc, idx_dec):
    enc_per_w = idx_enc.shape[0] // NW
    dec_per_w = idx_dec.shape[0] // NW
    mesh = plsc.VectorSubcoreMesh(
        core_axis_name="c", subcore_axis_name="s",
        num_cores=_NC, num_subcores=_NS)

    @functools.partial(
        pl.kernel,
        mesh=mesh,
        out_type=[
            jax.ShapeDtypeStruct((idx_enc.shape[0], EMB), jnp.float32),
            jax.ShapeDtypeStruct((idx_dec.shape[0], EMB), jnp.float32),
        ],
        scratch_types=[
            pltpu.VMEM((enc_per_w,), jnp.int32),
            pltpu.VMEM((enc_per_w, EMB), jnp.float32),
            pltpu.VMEM((dec_per_w,), jnp.int32),
            pltpu.VMEM((dec_per_w, EMB), jnp.float32),
            pltpu.SemaphoreType.DMA,
        ],
    )
    def k(table_hbm, ie_hbm, id_hbm, oe_hbm, od_hbm, ie_v, re_v, id_v, rd_v, sem):
        wid = lax.axis_index("s") * _NC + lax.axis_index("c")
        be = wid * enc_per_w
        bd = wid * dec_per_w
        pltpu.sync_copy(ie_hbm.at[pl.ds(be, enc_per_w)], ie_v)
        pltpu.async_copy(table_hbm.at[ie_v], re_v, sem).wait()
        pltpu.sync_copy(re_v, oe_hbm.at[pl.ds(be, enc_per_w)])
        pltpu.sync_copy(id_hbm.at[pl.ds(bd, dec_per_w)], id_v)
        pltpu.async_copy(table_hbm.at[id_v], rd_v, sem).wait()
        pltpu.sync_copy(rd_v, od_hbm.at[pl.ds(bd, dec_per_w)])

    return k(table, idx_enc, idx_dec)


def _tc_core_body(
    xs_ref, de_ref, scat_ref,
    wihf_ref, whhf_ref, bf_ref, wihb_ref, whhb_ref, bb_ref,
    wihd_ref, whhd_ref, bd_ref,
    u1_ref, u2_ref, q_ref,
    whw_ref, wsw_ref, wxw_ref, pgb_ref,
    v1w_ref, v1b_ref,
    hidden_ref, vals_ref, idx2_ref, pg_ref, cov_ref,
    hsf, hsb, hall,
):
    def cell(x, h, c, wih, whh, bias):
        gg = jnp.dot(x, wih, preferred_element_type=jnp.float32)
        gg = gg + jnp.dot(h, whh, preferred_element_type=jnp.float32) + bias
        ii = jax.nn.sigmoid(gg[:, 0:HID])
        ff = jax.nn.sigmoid(gg[:, HID:2 * HID])
        cc = jnp.tanh(gg[:, 2 * HID:3 * HID])
        oo = jax.nn.sigmoid(gg[:, 3 * HID:4 * HID])
        c2 = ff * c + ii * cc
        return oo * jnp.tanh(c2), c2

    wihf = wihf_ref[...]
    whhf = whhf_ref[...]
    bfv = bf_ref[...]
    wihb = wihb_ref[...]
    whhb = whhb_ref[...]
    bbv = bb_ref[...]

    def enc_step(t, carry):
        hf, cf, hb, cb = carry
        xf = xs_ref[t]
        xb = xs_ref[T - 1 - t]
        hf2, cf2 = cell(xf, hf, cf, wihf, whhf, bfv)
        hb2, cb2 = cell(xb, hb, cb, wihb, whhb, bbv)
        hsf[t] = hf2
        hsb[T - 1 - t] = hb2
        return hf2, cf2, hb2, cb2

    z = jnp.zeros((B, HID), jnp.float32)
    lax.fori_loop(0, T, enc_step, (z, z, z, z), unroll=2)

    wihd = wihd_ref[...]
    whhd = whhd_ref[...]
    bdv = bd_ref[...]

    def dec_step(i, carry):
        hd, cd = carry
        hd2, cd2 = cell(de_ref[i], hd, cd, wihd, whhd, bdv)
        hall[i] = hd2
        return hd2, cd2

    lax.fori_loop(0, NSTEP, dec_step, (z, z), unroll=4)

    hsf_all = hsf[...]
    hsb_all = hsb[...]
    hall_all = hall[...]

    u1 = u1_ref[...].reshape(1, 1, HID)
    u2 = u2_ref[...].reshape(1, 1, HID)
    qv = q_ref[...].reshape(1, 1, HID)

    E_tb = jnp.sum(hsf_all * u1, axis=-1) + jnp.sum(hsb_all * u2, axis=-1)
    w_tb = jnp.exp(E_tb - jnp.max(E_tb, axis=0, keepdims=True))

    s_ib = jnp.sum(hall_all * qv, axis=-1)
    g = jnp.exp(s_ib - jnp.max(s_ib, axis=1, keepdims=True))

    wv = w_tb.reshape(4, 32, B)
    W4 = jnp.sum(wv, axis=0)
    Z = jnp.dot(g, W4, preferred_element_type=jnp.float32)

    hfv = hsf_all.reshape(4, 32, B, HID)
    hbv = hsb_all.reshape(4, 32, B, HID)
    Pf = jnp.sum(wv[..., None] * hfv, axis=0)
    Pb = jnp.sum(wv[..., None] * hbv, axis=0)
    gb = g[:, :, None, None]
    ctx_f = jnp.sum(gb * Pf[None], axis=1) / Z[:, :, None]
    ctx_b = jnp.sum(gb * Pb[None], axis=1) / Z[:, :, None]

    g_exp = jnp.concatenate([g, g, g, g], axis=1)
    w_bt = w_tb.T
    attn = w_bt[None] * g_exp[:, None, :] / Z[:, :, None]

    cov = jnp.zeros((B, T), jnp.float32)
    cl = jnp.zeros((), jnp.float32)
    for i in range(NSTEP):
        cl = cl + jnp.sum(jnp.minimum(attn[i], cov))
        cov = cov + attn[i]
    cov_ref[...] = cl.reshape(1, 1)

    demb = de_ref[0:NSTEP]
    whw = whw_ref[...]
    pg_lin = (
        jnp.sum(ctx_f * whw[:, 0:HID].reshape(1, 1, HID), axis=-1)
        + jnp.sum(ctx_b * whw[:, HID:2 * HID].reshape(1, 1, HID), axis=-1)
        + jnp.sum(hall_all * wsw_ref[...].reshape(1, 1, HID), axis=-1)
        + jnp.sum(demb * wxw_ref[...].reshape(1, 1, EMB), axis=-1)
        + pgb_ref[0, 0]
    )
    pg = jax.nn.sigmoid(pg_lin)
    pg_ref[...] = pg
    valsv = (1.0 - pg)[:, :, None] * attn

    sidx = scat_ref[...]
    eqf = (sidx[:, :, None] == sidx[:, None, :]).astype(jnp.float32)
    t_i = lax.broadcasted_iota(jnp.int32, (T, T), 0)
    t_j = lax.broadcasted_iota(jnp.int32, (T, T), 1)
    tri = (t_j < t_i).astype(jnp.float32)
    first = jnp.sum(eqf * tri[None], axis=2) == 0.0
    firstf = first.astype(jnp.float32)
    v2 = [jnp.sum(eqf * valsv[i][:, None, :], axis=2) for i in range(NSTEP)]
    vals_ref[...] = jnp.stack(v2) * firstf[None]
    idx2_ref[...] = jnp.where(first, sidx, VPAD - 1)

    cat = jnp.concatenate([hall_all, ctx_f, ctx_b], axis=-1).reshape(NSTEP * B, 3 * HID)
    hidden_ref[...] = (
        jnp.dot(cat, v1w_ref[...], preferred_element_type=jnp.float32) + v1b_ref[...]
    )


def _tc_core(xs, de, scat_idx, wihf, whhf, bf, wihb, whhb, bb, wihd, whhd, bd,
             u1, u2, q, whw, wsw, wxw, pgb, v1w, v1b):
    return pl.pallas_call(
        _tc_core_body,
        out_shape=[
            jax.ShapeDtypeStruct((NSTEP * B, 3 * HID), jnp.float32),
            jax.ShapeDtypeStruct((NSTEP, B, T), jnp.float32),
            jax.ShapeDtypeStruct((B, T), jnp.int32),
            jax.ShapeDtypeStruct((NSTEP, B), jnp.float32),
            jax.ShapeDtypeStruct((1, 1), jnp.float32),
        ],
        scratch_shapes=[
            pltpu.VMEM((T, B, HID), jnp.float32),
            pltpu.VMEM((T, B, HID), jnp.float32),
            pltpu.VMEM((NSTEP, B, HID), jnp.float32),
        ],
    )(xs, de, scat_idx, wihf, whhf, bf, wihb, whhb, bb, wihd, whhd, bd,
      u1, u2, q, whw, wsw, wxw, pgb, v1w, v1b)


def _tc_logits_body(h_ref, v2_ref, b2_ref, out_ref):
    out_ref[...] = (
        jnp.dot(h_ref[...], v2_ref[...], preferred_element_type=jnp.float32)
        + b2_ref[...]
    )


def _tc_logits(hidden, v2t_pad, b2_pad):
    return pl.pallas_call(
        _tc_logits_body,
        grid=(NVT,),
        in_specs=[
            pl.BlockSpec((NSTEP * B, 3 * HID), lambda j: (0, 0)),
            pl.BlockSpec((3 * HID, VTILE), lambda j: (0, j)),
            pl.BlockSpec((1, VTILE), lambda j: (0, j)),
        ],
        out_specs=pl.BlockSpec((NSTEP * B, VTILE), lambda j: (0, j)),
        out_shape=jax.ShapeDtypeStruct((NSTEP * B, VPAD), jnp.float32),
    )(hidden, v2t_pad, b2_pad)


def _tc_pev_body(l_ref, pg_ref, out_ref):
    i = pl.program_id(0)
    l = l_ref[...]
    m = jnp.max(l, axis=1, keepdims=True)
    e = jnp.exp(l - m)
    pv = e / jnp.sum(e, axis=1, keepdims=True)
    pgi = jnp.transpose(pg_ref[pl.ds(i, 1), :])
    out_ref[...] = 1.0 / WORD_COUNT / 100.0 + pgi * pv


def _tc_pev(logits, pg):
    return pl.pallas_call(
        _tc_pev_body,
        grid=(NSTEP,),
        in_specs=[
            pl.BlockSpec((B, VPAD), lambda i: (i, 0)),
            pl.BlockSpec((NSTEP, B), lambda i: (0, 0)),
        ],
        out_specs=pl.BlockSpec((B, VPAD), lambda i: (0, i)),
        out_shape=jax.ShapeDtypeStruct((B, NSTEP * VPAD), jnp.float32),
    )(logits, pg)


def _sc_scatter(pev, vals, scat_idx):
    mesh = plsc.VectorSubcoreMesh(
        core_axis_name="c", subcore_axis_name="s",
        num_cores=_NC, num_subcores=_NS)

    @functools.partial(
        pl.kernel,
        mesh=mesh,
        out_type=jax.ShapeDtypeStruct((B, NSTEP, VOUT), jnp.float32),
        scratch_types=[
            pltpu.VMEM((1, VOUT), jnp.float32),
            pltpu.VMEM((T,), jnp.int32),
            pltpu.VMEM((T,), jnp.float32),
        ],
        compiler_params=pltpu.CompilerParams(needs_layout_passes=False),
    )
    def k(pev_hbm, vals_hbm, idx_hbm, out_hbm, row_v, idx_v, val_v):
        b = lax.axis_index("s") * _NC + lax.axis_index("c")
        pltpu.sync_copy(idx_hbm.at[pl.ds(b * T, T)], idx_v)
        z16 = jnp.zeros((16,), jnp.int32)
        base16 = jnp.full((16,), 1.0 / WORD_COUNT / 100.0, jnp.float32)
        for i in range(NSTEP):
            pltpu.sync_copy(
                pev_hbm.at[pl.ds(b, 1), pl.ds(i * VPAD, VHEAD)],
                row_v.at[:, pl.ds(0, VHEAD)],
            )
            for off in (VHEAD, VHEAD + 16, VHEAD + 32, VOUT - 16):
                row_v[0, pl.ds(off, 16)] = base16
            pltpu.sync_copy(vals_hbm.at[pl.ds((i * B + b) * T, T)], val_v)
            for gk in range(T // 16):
                iv = idx_v[pl.ds(gk * 16, 16)]
                vv = val_v[pl.ds(gk * 16, 16)]
                plsc.addupdate_scatter(row_v, [z16, iv], vv,
                                       mask=iv < WORD_COUNT)
            pltpu.sync_copy(row_v, out_hbm.at[b].at[pl.ds(i, 1), :])

    return k(pev, vals.reshape(-1), scat_idx.reshape(-1))


def kernel(inputs, target, params):
    p = params
    unked = jnp.where(inputs >= WORD_COUNT, 1, inputs)
    idx_enc = unked.T.reshape(-1)
    idx_dec = jnp.where(target >= WORD_COUNT, 1, target).T.reshape(-1)
    scat_idx = inputs.reshape(-1).reshape(T, B).T

    enc_emb, dec_emb = _sc_gather(p["embed"], idx_enc, idx_dec)
    xs = enc_emb.reshape(T, B, EMB)
    de = dec_emb.reshape(TL, B, EMB)

    v_w = p["v_w"][0]
    u = p["Wh_w"].T @ v_w
    q = (p["Ws_w"].T @ v_w).reshape(1, HID)
    u1 = u[:HID].reshape(1, HID)
    u2 = u[HID:].reshape(1, HID)
    pg_bias = (p["wh_b"] + p["ws_b"] + p["wx_b"]).reshape(1, 1)

    hidden, vals, idx2, pg, cov = _tc_core(
        xs, de, scat_idx,
        p["enc_Wih_f"].T, p["enc_Whh_f"].T, (p["enc_bih_f"] + p["enc_bhh_f"]).reshape(1, -1),
        p["enc_Wih_b"].T, p["enc_Whh_b"].T, (p["enc_bih_b"] + p["enc_bhh_b"]).reshape(1, -1),
        p["dec_Wih"].T, p["dec_Whh"].T, (p["dec_bih"] + p["dec_bhh"]).reshape(1, -1),
        u1, u2, q,
        p["wh_w"], p["ws_w"].reshape(1, HID), p["wx_w"].reshape(1, EMB), pg_bias,
        p["V1_w"].T, p["V1_b"].reshape(1, -1),
    )

    v2t = jnp.pad(p["V2_w"].T, ((0, 0), (0, VPAD - WORD_COUNT)))
    b2 = jnp.pad(p["V2_b"], (0, VPAD - WORD_COUNT),
                 constant_values=-1e30).reshape(1, VPAD)

    logits = _tc_logits(hidden, v2t, b2)
    pev = _tc_pev(logits, pg)
    out = _sc_scatter(pev, vals, idx2)

    return out, cov.reshape(())


# speedup vs baseline: 3.8534x; 1.0003x over previous
import functools

import jax
import jax.numpy as jnp
from jax import lax
from jax.experimental import pallas as pl
from jax.experimental.pallas import tpu as pltpu
from jax.experimental.pallas import tpu_sc as plsc

WORD_COUNT = 50000
EMB = 128
HID = 256
MAX_OOVS = 100
B = 32
T = 128
TL = 16
NSTEP = TL - 1
VPAD = 50176
VTILE = 1024
NVT = VPAD // VTILE
VOUT = WORD_COUNT + MAX_OOVS
VHEAD = 50048

_NC, _NS = 2, 16
NW = _NC * _NS


def _sc_gather(table, idx_enc, idx_dec):
    enc_per_w = idx_enc.shape[0] // NW
    dec_per_w = idx_dec.shape[0] // NW
    mesh = plsc.VectorSubcoreMesh(
        core_axis_name="c", subcore_axis_name="s",
        num_cores=_NC, num_subcores=_NS)

    @functools.partial(
        pl.kernel,
        mesh=mesh,
        out_type=[
            jax.ShapeDtypeStruct((idx_enc.shape[0], EMB), jnp.float32),
            jax.ShapeDtypeStruct((idx_dec.shape[0], EMB), jnp.float32),
        ],
        scratch_types=[
            pltpu.VMEM((enc_per_w,), jnp.int32),
            pltpu.VMEM((enc_per_w, EMB), jnp.float32),
            pltpu.VMEM((dec_per_w,), jnp.int32),
            pltpu.VMEM((dec_per_w, EMB), jnp.float32),
            pltpu.SemaphoreType.DMA,
        ],
    )
    def k(table_hbm, ie_hbm, id_hbm, oe_hbm, od_hbm, ie_v, re_v, id_v, rd_v, sem):
        wid = lax.axis_index("s") * _NC + lax.axis_index("c")
        be = wid * enc_per_w
        bd = wid * dec_per_w
        pltpu.sync_copy(ie_hbm.at[pl.ds(be, enc_per_w)], ie_v)
        pltpu.async_copy(table_hbm.at[ie_v], re_v, sem).wait()
        pltpu.sync_copy(re_v, oe_hbm.at[pl.ds(be, enc_per_w)])
        pltpu.sync_copy(id_hbm.at[pl.ds(bd, dec_per_w)], id_v)
        pltpu.async_copy(table_hbm.at[id_v], rd_v, sem).wait()
        pltpu.sync_copy(rd_v, od_hbm.at[pl.ds(bd, dec_per_w)])

    return k(table, idx_enc, idx_dec)


def _tc_core_body(
    xs_ref, de_ref, scat_ref,
    wihf_ref, whhf_ref, bf_ref, wihb_ref, whhb_ref, bb_ref,
    wihd_ref, whhd_ref, bd_ref,
    u1_ref, u2_ref, q_ref,
    whw_ref, wsw_ref, wxw_ref, pgb_ref,
    v1w_ref, v1b_ref,
    hidden_ref, vals_ref, idx2_ref, pg_ref, cov_ref,
    hsf, hsb, hall,
):
    def cell(x, h, c, wih, whh, bias):
        gg = jnp.dot(x, wih, preferred_element_type=jnp.float32)
        gg = gg + jnp.dot(h, whh, preferred_element_type=jnp.float32) + bias
        ii = jax.nn.sigmoid(gg[:, 0:HID])
        ff = jax.nn.sigmoid(gg[:, HID:2 * HID])
        cc = jnp.tanh(gg[:, 2 * HID:3 * HID])
        oo = jax.nn.sigmoid(gg[:, 3 * HID:4 * HID])
        c2 = ff * c + ii * cc
        return oo * jnp.tanh(c2), c2

    wihf = wihf_ref[...]
    whhf = whhf_ref[...]
    bfv = bf_ref[...]
    wihb = wihb_ref[...]
    whhb = whhb_ref[...]
    bbv = bb_ref[...]

    def enc_step(t, carry):
        hf, cf, hb, cb = carry
        xf = xs_ref[t]
        xb = xs_ref[T - 1 - t]
        hf2, cf2 = cell(xf, hf, cf, wihf, whhf, bfv)
        hb2, cb2 = cell(xb, hb, cb, wihb, whhb, bbv)
        hsf[t] = hf2
        hsb[T - 1 - t] = hb2
        return hf2, cf2, hb2, cb2

    z = jnp.zeros((B, HID), jnp.float32)
    lax.fori_loop(0, T, enc_step, (z, z, z, z), unroll=2)

    wihd = wihd_ref[...]
    whhd = whhd_ref[...]
    bdv = bd_ref[...]

    def dec_step(i, carry):
        hd, cd = carry
        hd2, cd2 = cell(de_ref[i], hd, cd, wihd, whhd, bdv)
        hall[i] = hd2
        return hd2, cd2

    lax.fori_loop(0, NSTEP, dec_step, (z, z), unroll=4)

    hsf_all = hsf[...]
    hsb_all = hsb[...]
    hall_all = hall[...]

    u1 = u1_ref[...].reshape(1, 1, HID)
    u2 = u2_ref[...].reshape(1, 1, HID)
    qv = q_ref[...].reshape(1, 1, HID)

    E_tb = jnp.sum(hsf_all * u1, axis=-1) + jnp.sum(hsb_all * u2, axis=-1)
    w_tb = jnp.exp(E_tb - jnp.max(E_tb, axis=0, keepdims=True))

    s_ib = jnp.sum(hall_all * qv, axis=-1)
    g = jnp.exp(s_ib - jnp.max(s_ib, axis=1, keepdims=True))

    wv = w_tb.reshape(4, 32, B)
    W4 = jnp.sum(wv, axis=0)
    Z = jnp.dot(g, W4, preferred_element_type=jnp.float32)

    hfv = hsf_all.reshape(4, 32, B, HID)
    hbv = hsb_all.reshape(4, 32, B, HID)
    Pf = jnp.sum(wv[..., None] * hfv, axis=0)
    Pb = jnp.sum(wv[..., None] * hbv, axis=0)
    gb = g[:, :, None, None]
    ctx_f = jnp.sum(gb * Pf[None], axis=1) / Z[:, :, None]
    ctx_b = jnp.sum(gb * Pb[None], axis=1) / Z[:, :, None]

    g_exp = jnp.concatenate([g, g, g, g], axis=1)
    w_bt = w_tb.T
    attn = w_bt[None] * g_exp[:, None, :] / Z[:, :, None]

    cov = jnp.zeros((B, T), jnp.float32)
    cl = jnp.zeros((), jnp.float32)
    for i in range(NSTEP):
        cl = cl + jnp.sum(jnp.minimum(attn[i], cov))
        cov = cov + attn[i]
    cov_ref[...] = cl.reshape(1, 1)

    demb = de_ref[0:NSTEP]
    whw = whw_ref[...]
    pg_lin = (
        jnp.sum(ctx_f * whw[:, 0:HID].reshape(1, 1, HID), axis=-1)
        + jnp.sum(ctx_b * whw[:, HID:2 * HID].reshape(1, 1, HID), axis=-1)
        + jnp.sum(hall_all * wsw_ref[...].reshape(1, 1, HID), axis=-1)
        + jnp.sum(demb * wxw_ref[...].reshape(1, 1, EMB), axis=-1)
        + pgb_ref[0, 0]
    )
    pg = jax.nn.sigmoid(pg_lin)
    pg_ref[...] = pg
    valsv = (1.0 - pg)[:, :, None] * attn

    sidx = scat_ref[...]
    eqf = (sidx[:, :, None] == sidx[:, None, :]).astype(jnp.float32)
    t_i = lax.broadcasted_iota(jnp.int32, (T, T), 0)
    t_j = lax.broadcasted_iota(jnp.int32, (T, T), 1)
    tri = (t_j < t_i).astype(jnp.float32)
    first = jnp.sum(eqf * tri[None], axis=2) == 0.0
    firstf = first.astype(jnp.float32)
    v2 = [jnp.sum(eqf * valsv[i][:, None, :], axis=2) for i in range(NSTEP)]
    vals_ref[...] = jnp.stack(v2) * firstf[None]
    idx2_ref[...] = jnp.where(first, sidx, VPAD - 1)

    cat = jnp.concatenate([hall_all, ctx_f, ctx_b], axis=-1).reshape(NSTEP * B, 3 * HID)
    hidden_ref[...] = (
        jnp.dot(cat, v1w_ref[...], preferred_element_type=jnp.float32) + v1b_ref[...]
    )


def _tc_core(xs, de, scat_idx, wihf, whhf, bf, wihb, whhb, bb, wihd, whhd, bd,
             u1, u2, q, whw, wsw, wxw, pgb, v1w, v1b):
    return pl.pallas_call(
        _tc_core_body,
        out_shape=[
            jax.ShapeDtypeStruct((NSTEP * B, 3 * HID), jnp.float32),
            jax.ShapeDtypeStruct((NSTEP, B, T), jnp.float32),
            jax.ShapeDtypeStruct((B, T), jnp.int32),
            jax.ShapeDtypeStruct((NSTEP, B), jnp.float32),
            jax.ShapeDtypeStruct((1, 1), jnp.float32),
        ],
        scratch_shapes=[
            pltpu.VMEM((T, B, HID), jnp.float32),
            pltpu.VMEM((T, B, HID), jnp.float32),
            pltpu.VMEM((NSTEP, B, HID), jnp.float32),
        ],
    )(xs, de, scat_idx, wihf, whhf, bf, wihb, whhb, bb, wihd, whhd, bd,
      u1, u2, q, whw, wsw, wxw, pgb, v1w, v1b)


def _tc_logits_body(h_ref, v2_ref, b2_ref, out_ref):
    h_bf = h_ref[...].astype(jnp.bfloat16)
    v2_bf = v2_ref[...].astype(jnp.bfloat16)
    out_ref[...] = (
        jnp.dot(h_bf, v2_bf, preferred_element_type=jnp.float32)
        + b2_ref[...]
    )


def _tc_logits(hidden, v2t_pad, b2_pad):
    return pl.pallas_call(
        _tc_logits_body,
        grid=(NVT,),
        in_specs=[
            pl.BlockSpec((NSTEP * B, 3 * HID), lambda j: (0, 0)),
            pl.BlockSpec((3 * HID, VTILE), lambda j: (0, j)),
            pl.BlockSpec((1, VTILE), lambda j: (0, j)),
        ],
        out_specs=pl.BlockSpec((NSTEP * B, VTILE), lambda j: (0, j)),
        out_shape=jax.ShapeDtypeStruct((NSTEP * B, VPAD), jnp.float32),
    )(hidden, v2t_pad, b2_pad)


def _tc_pev_body(l_ref, pg_ref, out_ref):
    i = pl.program_id(0)
    l = l_ref[...]
    m = jnp.max(l, axis=1, keepdims=True)
    e = jnp.exp(l - m)
    pv = e / jnp.sum(e, axis=1, keepdims=True)
    pgi = jnp.transpose(pg_ref[pl.ds(i, 1), :])
    out_ref[...] = 1.0 / WORD_COUNT / 100.0 + pgi * pv


def _tc_pev(logits, pg):
    return pl.pallas_call(
        _tc_pev_body,
        grid=(NSTEP,),
        in_specs=[
            pl.BlockSpec((B, VPAD), lambda i: (i, 0)),
            pl.BlockSpec((NSTEP, B), lambda i: (0, 0)),
        ],
        out_specs=pl.BlockSpec((B, VPAD), lambda i: (0, i)),
        out_shape=jax.ShapeDtypeStruct((B, NSTEP * VPAD), jnp.float32),
    )(logits, pg)


def _sc_scatter(pev, vals, scat_idx):
    mesh = plsc.VectorSubcoreMesh(
        core_axis_name="c", subcore_axis_name="s",
        num_cores=_NC, num_subcores=_NS)

    @functools.partial(
        pl.kernel,
        mesh=mesh,
        out_type=jax.ShapeDtypeStruct((B, NSTEP, VOUT), jnp.float32),
        scratch_types=[
            pltpu.VMEM((1, VOUT), jnp.float32),
            pltpu.VMEM((T,), jnp.int32),
            pltpu.VMEM((T,), jnp.float32),
        ],
        compiler_params=pltpu.CompilerParams(
            needs_layout_passes=False, use_tc_tiling_on_sc=True),
    )
    def k(pev_hbm, vals_hbm, idx_hbm, out_hbm, row_v, idx_v, val_v):
        b = lax.axis_index("s") * _NC + lax.axis_index("c")
        pltpu.sync_copy(idx_hbm.at[pl.ds(b * T, T)], idx_v)
        z16 = jnp.zeros((16,), jnp.int32)
        base16 = jnp.full((16,), 1.0 / WORD_COUNT / 100.0, jnp.float32)
        for i in range(NSTEP):
            pltpu.sync_copy(
                pev_hbm.at[pl.ds(b, 1), pl.ds(i * VPAD, VHEAD)],
                row_v.at[:, pl.ds(0, VHEAD)],
            )
            for off in (VHEAD, VHEAD + 16, VHEAD + 32, VOUT - 16):
                row_v[0, pl.ds(off, 16)] = base16
            pltpu.sync_copy(vals_hbm.at[pl.ds((i * B + b) * T, T)], val_v)
            for gk in range(T // 16):
                iv = idx_v[pl.ds(gk * 16, 16)]
                vv = val_v[pl.ds(gk * 16, 16)]
                plsc.addupdate_scatter(row_v, [z16, iv], vv,
                                       mask=iv < WORD_COUNT)
            pltpu.sync_copy(row_v, out_hbm.at[b].at[pl.ds(i, 1), :])

    return k(pev, vals.reshape(-1), scat_idx.reshape(-1))


def kernel(inputs, target, params):
    p = params
    unked = jnp.where(inputs >= WORD_COUNT, 1, inputs)
    idx_enc = unked.T.reshape(-1)
    idx_dec = jnp.where(target >= WORD_COUNT, 1, target).T.reshape(-1)
    scat_idx = inputs.reshape(-1).reshape(T, B).T

    enc_emb, dec_emb = _sc_gather(p["embed"], idx_enc, idx_dec)
    xs = enc_emb.reshape(T, B, EMB)
    de = dec_emb.reshape(TL, B, EMB)

    v_w = p["v_w"][0]
    u = p["Wh_w"].T @ v_w
    q = (p["Ws_w"].T @ v_w).reshape(1, HID)
    u1 = u[:HID].reshape(1, HID)
    u2 = u[HID:].reshape(1, HID)
    pg_bias = (p["wh_b"] + p["ws_b"] + p["wx_b"]).reshape(1, 1)

    hidden, vals, idx2, pg, cov = _tc_core(
        xs, de, scat_idx,
        p["enc_Wih_f"].T, p["enc_Whh_f"].T, (p["enc_bih_f"] + p["enc_bhh_f"]).reshape(1, -1),
        p["enc_Wih_b"].T, p["enc_Whh_b"].T, (p["enc_bih_b"] + p["enc_bhh_b"]).reshape(1, -1),
        p["dec_Wih"].T, p["dec_Whh"].T, (p["dec_bih"] + p["dec_bhh"]).reshape(1, -1),
        u1, u2, q,
        p["wh_w"], p["ws_w"].reshape(1, HID), p["wx_w"].reshape(1, EMB), pg_bias,
        p["V1_w"].T, p["V1_b"].reshape(1, -1),
    )

    v2t = jnp.pad(p["V2_w"].T, ((0, 0), (0, VPAD - WORD_COUNT)))
    b2 = jnp.pad(p["V2_b"], (0, VPAD - WORD_COUNT),
                 constant_values=-1e30).reshape(1, VPAD)

    logits = _tc_logits(hidden, v2t, b2)
    pev = _tc_pev(logits, pg)
    out = _sc_scatter(pev, vals, idx2)

    return out, cov.reshape(())


# speedup vs baseline: 4.4161x; 1.1460x over previous
import functools

import jax
import jax.numpy as jnp
from jax import lax
from jax.experimental import pallas as pl
from jax.experimental.pallas import tpu as pltpu
from jax.experimental.pallas import tpu_sc as plsc

WORD_COUNT = 50000
EMB = 128
HID = 256
MAX_OOVS = 100
B = 32
T = 128
TL = 16
NSTEP = TL - 1
VPAD = 50176
VTILE = 1024
NVT = VPAD // VTILE
VOUT = WORD_COUNT + MAX_OOVS
VHEAD = 50048

_NC, _NS = 2, 16
NW = _NC * _NS


def _sc_gather(table, idx_enc, idx_dec):
    enc_per_w = idx_enc.shape[0] // NW
    dec_per_w = idx_dec.shape[0] // NW
    mesh = plsc.VectorSubcoreMesh(
        core_axis_name="c", subcore_axis_name="s",
        num_cores=_NC, num_subcores=_NS)

    @functools.partial(
        pl.kernel,
        mesh=mesh,
        out_type=[
            jax.ShapeDtypeStruct((idx_enc.shape[0], EMB), jnp.float32),
            jax.ShapeDtypeStruct((idx_dec.shape[0], EMB), jnp.float32),
        ],
        scratch_types=[
            pltpu.VMEM((enc_per_w,), jnp.int32),
            pltpu.VMEM((enc_per_w, EMB), jnp.float32),
            pltpu.VMEM((dec_per_w,), jnp.int32),
            pltpu.VMEM((dec_per_w, EMB), jnp.float32),
            pltpu.SemaphoreType.DMA,
        ],
    )
    def k(table_hbm, ie_hbm, id_hbm, oe_hbm, od_hbm, ie_v, re_v, id_v, rd_v, sem):
        wid = lax.axis_index("s") * _NC + lax.axis_index("c")
        be = wid * enc_per_w
        bd = wid * dec_per_w
        pltpu.sync_copy(ie_hbm.at[pl.ds(be, enc_per_w)], ie_v)
        pltpu.async_copy(table_hbm.at[ie_v], re_v, sem).wait()
        pltpu.sync_copy(re_v, oe_hbm.at[pl.ds(be, enc_per_w)])
        pltpu.sync_copy(id_hbm.at[pl.ds(bd, dec_per_w)], id_v)
        pltpu.async_copy(table_hbm.at[id_v], rd_v, sem).wait()
        pltpu.sync_copy(rd_v, od_hbm.at[pl.ds(bd, dec_per_w)])

    return k(table, idx_enc, idx_dec)


def _tc_core_body(
    xs_ref, de_ref, scat_ref,
    wihf_ref, whhf_ref, bf_ref, wihb_ref, whhb_ref, bb_ref,
    wihd_ref, whhd_ref, bd_ref,
    u1_ref, u2_ref, q_ref,
    whw_ref, wsw_ref, wxw_ref, pgb_ref,
    v1w_ref, v1b_ref,
    hidden_ref, vals_ref, idx2_ref, pg_ref, cov_ref,
    hsf, hsb, hall,
):
    def cell(x, h, c, wih, whh, bias):
        gg = jnp.dot(x, wih, preferred_element_type=jnp.float32)
        gg = gg + jnp.dot(h, whh, preferred_element_type=jnp.float32) + bias
        ii = jax.nn.sigmoid(gg[:, 0:HID])
        ff = jax.nn.sigmoid(gg[:, HID:2 * HID])
        cc = jnp.tanh(gg[:, 2 * HID:3 * HID])
        oo = jax.nn.sigmoid(gg[:, 3 * HID:4 * HID])
        c2 = ff * c + ii * cc
        return oo * jnp.tanh(c2), c2

    wihf = wihf_ref[...]
    whhf = whhf_ref[...]
    bfv = bf_ref[...]
    wihb = wihb_ref[...]
    whhb = whhb_ref[...]
    bbv = bb_ref[...]

    def enc_step(t, carry):
        hf, cf, hb, cb = carry
        xf = xs_ref[t]
        xb = xs_ref[T - 1 - t]
        hf2, cf2 = cell(xf, hf, cf, wihf, whhf, bfv)
        hb2, cb2 = cell(xb, hb, cb, wihb, whhb, bbv)
        hsf[t] = hf2
        hsb[T - 1 - t] = hb2
        return hf2, cf2, hb2, cb2

    z = jnp.zeros((B, HID), jnp.float32)
    lax.fori_loop(0, T, enc_step, (z, z, z, z), unroll=2)

    wihd = wihd_ref[...]
    whhd = whhd_ref[...]
    bdv = bd_ref[...]

    def dec_step(i, carry):
        hd, cd = carry
        hd2, cd2 = cell(de_ref[i], hd, cd, wihd, whhd, bdv)
        hall[i] = hd2
        return hd2, cd2

    lax.fori_loop(0, NSTEP, dec_step, (z, z), unroll=4)

    hsf_all = hsf[...]
    hsb_all = hsb[...]
    hall_all = hall[...]

    u1 = u1_ref[...].reshape(1, 1, HID)
    u2 = u2_ref[...].reshape(1, 1, HID)
    qv = q_ref[...].reshape(1, 1, HID)

    E_tb = jnp.sum(hsf_all * u1, axis=-1) + jnp.sum(hsb_all * u2, axis=-1)
    w_tb = jnp.exp(E_tb - jnp.max(E_tb, axis=0, keepdims=True))

    s_ib = jnp.sum(hall_all * qv, axis=-1)
    g = jnp.exp(s_ib - jnp.max(s_ib, axis=1, keepdims=True))

    wv = w_tb.reshape(4, 32, B)
    W4 = jnp.sum(wv, axis=0)
    Z = jnp.dot(g, W4, preferred_element_type=jnp.float32)

    hfv = hsf_all.reshape(4, 32, B, HID)
    hbv = hsb_all.reshape(4, 32, B, HID)
    Pf = jnp.sum(wv[..., None] * hfv, axis=0)
    Pb = jnp.sum(wv[..., None] * hbv, axis=0)
    gb = g[:, :, None, None]
    ctx_f = jnp.sum(gb * Pf[None], axis=1) / Z[:, :, None]
    ctx_b = jnp.sum(gb * Pb[None], axis=1) / Z[:, :, None]

    g_exp = jnp.concatenate([g, g, g, g], axis=1)
    w_bt = w_tb.T
    attn = w_bt[None] * g_exp[:, None, :] / Z[:, :, None]

    cov = jnp.zeros((B, T), jnp.float32)
    cl = jnp.zeros((), jnp.float32)
    for i in range(NSTEP):
        cl = cl + jnp.sum(jnp.minimum(attn[i], cov))
        cov = cov + attn[i]
    cov_ref[...] = cl.reshape(1, 1)

    demb = de_ref[0:NSTEP]
    whw = whw_ref[...]
    pg_lin = (
        jnp.sum(ctx_f * whw[:, 0:HID].reshape(1, 1, HID), axis=-1)
        + jnp.sum(ctx_b * whw[:, HID:2 * HID].reshape(1, 1, HID), axis=-1)
        + jnp.sum(hall_all * wsw_ref[...].reshape(1, 1, HID), axis=-1)
        + jnp.sum(demb * wxw_ref[...].reshape(1, 1, EMB), axis=-1)
        + pgb_ref[0, 0]
    )
    pg = jax.nn.sigmoid(pg_lin)
    pg_ref[...] = pg
    valsv = (1.0 - pg)[:, :, None] * attn

    sidx = scat_ref[...]
    eqf = (sidx[:, :, None] == sidx[:, None, :]).astype(jnp.float32)
    t_i = lax.broadcasted_iota(jnp.int32, (T, T), 0)
    t_j = lax.broadcasted_iota(jnp.int32, (T, T), 1)
    tri = (t_j < t_i).astype(jnp.float32)
    first = jnp.sum(eqf * tri[None], axis=2) == 0.0
    firstf = first.astype(jnp.float32)
    v2 = [jnp.sum(eqf * valsv[i][:, None, :], axis=2) for i in range(NSTEP)]
    vals_ref[...] = jnp.stack(v2) * firstf[None]
    idx2_ref[...] = jnp.where(first, sidx, VPAD - 1)

    cat = jnp.concatenate([hall_all, ctx_f, ctx_b], axis=-1).reshape(NSTEP * B, 3 * HID)
    hidden_ref[...] = (
        jnp.dot(cat, v1w_ref[...], preferred_element_type=jnp.float32) + v1b_ref[...]
    )


def _tc_core(xs, de, scat_idx, wihf, whhf, bf, wihb, whhb, bb, wihd, whhd, bd,
             u1, u2, q, whw, wsw, wxw, pgb, v1w, v1b):
    return pl.pallas_call(
        _tc_core_body,
        out_shape=[
            jax.ShapeDtypeStruct((NSTEP * B, 3 * HID), jnp.float32),
            jax.ShapeDtypeStruct((NSTEP, B, T), jnp.float32),
            jax.ShapeDtypeStruct((B, T), jnp.int32),
            jax.ShapeDtypeStruct((NSTEP, B), jnp.float32),
            jax.ShapeDtypeStruct((1, 1), jnp.float32),
        ],
        scratch_shapes=[
            pltpu.VMEM((T, B, HID), jnp.float32),
            pltpu.VMEM((T, B, HID), jnp.float32),
            pltpu.VMEM((NSTEP, B, HID), jnp.float32),
        ],
    )(xs, de, scat_idx, wihf, whhf, bf, wihb, whhb, bb, wihd, whhd, bd,
      u1, u2, q, whw, wsw, wxw, pgb, v1w, v1b)


def _tc_logits_body(h_ref, v2_ref, b2_ref, out_ref):
    h_bf = h_ref[...].astype(jnp.bfloat16)
    v2_bf = v2_ref[...].astype(jnp.bfloat16)
    out_ref[...] = (
        jnp.dot(h_bf, v2_bf, preferred_element_type=jnp.float32)
        + b2_ref[...]
    )


def _tc_logits(hidden, v2t_pad, b2_pad):
    return pl.pallas_call(
        _tc_logits_body,
        grid=(NVT,),
        in_specs=[
            pl.BlockSpec((NSTEP * B, 3 * HID), lambda j: (0, 0)),
            pl.BlockSpec((3 * HID, VTILE), lambda j: (0, j)),
            pl.BlockSpec((1, VTILE), lambda j: (0, j)),
        ],
        out_specs=pl.BlockSpec((NSTEP * B, VTILE), lambda j: (0, j)),
        out_shape=jax.ShapeDtypeStruct((NSTEP * B, VPAD), jnp.float32),
    )(hidden, v2t_pad, b2_pad)


def _tc_pev_body(l_ref, pg_ref, out_ref):
    i = pl.program_id(0)
    l = l_ref[...]
    m = jnp.max(l, axis=1, keepdims=True)
    e = jnp.exp(l - m)
    pv = e / jnp.sum(e, axis=1, keepdims=True)
    pgi = jnp.transpose(pg_ref[pl.ds(i, 1), :])
    out_ref[...] = 1.0 / WORD_COUNT / 100.0 + pgi * pv


def _tc_pev(logits, pg):
    return pl.pallas_call(
        _tc_pev_body,
        grid=(NSTEP,),
        in_specs=[
            pl.BlockSpec((B, VPAD), lambda i: (i, 0)),
            pl.BlockSpec((NSTEP, B), lambda i: (0, 0)),
        ],
        out_specs=pl.BlockSpec((B, VPAD), lambda i: (0, i)),
        out_shape=jax.ShapeDtypeStruct((B, NSTEP * VPAD), jnp.float32),
    )(logits, pg)


def _sc_scatter(pev, vals, scat_idx):
    mesh = plsc.VectorSubcoreMesh(
        core_axis_name="c", subcore_axis_name="s",
        num_cores=_NC, num_subcores=_NS)

    @functools.partial(
        pl.kernel,
        mesh=mesh,
        out_type=jax.ShapeDtypeStruct((NSTEP, B, VOUT), jnp.float32),
        scratch_types=[
            pltpu.VMEM((1, VOUT), jnp.float32),
            pltpu.VMEM((T,), jnp.int32),
            pltpu.VMEM((T,), jnp.float32),
        ],
        compiler_params=pltpu.CompilerParams(
            needs_layout_passes=False, use_tc_tiling_on_sc=True),
    )
    def k(pev_hbm, vals_hbm, idx_hbm, out_hbm, row_v, idx_v, val_v):
        b = lax.axis_index("s") * _NC + lax.axis_index("c")
        pltpu.sync_copy(idx_hbm.at[pl.ds(b * T, T)], idx_v)
        z16 = jnp.zeros((16,), jnp.int32)
        base16 = jnp.full((16,), 1.0 / WORD_COUNT / 100.0, jnp.float32)
        for i in range(NSTEP):
            pltpu.sync_copy(
                pev_hbm.at[pl.ds(b, 1), pl.ds(i * VPAD, VHEAD)],
                row_v.at[:, pl.ds(0, VHEAD)],
            )
            for off in (VHEAD, VHEAD + 16, VHEAD + 32, VOUT - 16):
                row_v[0, pl.ds(off, 16)] = base16
            pltpu.sync_copy(vals_hbm.at[pl.ds((i * B + b) * T, T)], val_v)
            for gk in range(T // 16):
                iv = idx_v[pl.ds(gk * 16, 16)]
                vv = val_v[pl.ds(gk * 16, 16)]
                plsc.addupdate_scatter(row_v, [z16, iv], vv,
                                       mask=iv < WORD_COUNT)
            pltpu.sync_copy(row_v, out_hbm.at[i].at[pl.ds(b, 1), :])

    return k(pev, vals.reshape(-1), scat_idx.reshape(-1))


def kernel(inputs, target, params):
    p = params
    unked = jnp.where(inputs >= WORD_COUNT, 1, inputs)
    idx_enc = unked.T.reshape(-1)
    idx_dec = jnp.where(target >= WORD_COUNT, 1, target).T.reshape(-1)
    scat_idx = inputs.reshape(-1).reshape(T, B).T

    enc_emb, dec_emb = _sc_gather(p["embed"], idx_enc, idx_dec)
    xs = enc_emb.reshape(T, B, EMB)
    de = dec_emb.reshape(TL, B, EMB)

    v_w = p["v_w"][0]
    u = p["Wh_w"].T @ v_w
    q = (p["Ws_w"].T @ v_w).reshape(1, HID)
    u1 = u[:HID].reshape(1, HID)
    u2 = u[HID:].reshape(1, HID)
    pg_bias = (p["wh_b"] + p["ws_b"] + p["wx_b"]).reshape(1, 1)

    hidden, vals, idx2, pg, cov = _tc_core(
        xs, de, scat_idx,
        p["enc_Wih_f"].T, p["enc_Whh_f"].T, (p["enc_bih_f"] + p["enc_bhh_f"]).reshape(1, -1),
        p["enc_Wih_b"].T, p["enc_Whh_b"].T, (p["enc_bih_b"] + p["enc_bhh_b"]).reshape(1, -1),
        p["dec_Wih"].T, p["dec_Whh"].T, (p["dec_bih"] + p["dec_bhh"]).reshape(1, -1),
        u1, u2, q,
        p["wh_w"], p["ws_w"].reshape(1, HID), p["wx_w"].reshape(1, EMB), pg_bias,
        p["V1_w"].T, p["V1_b"].reshape(1, -1),
    )

    v2t = jnp.pad(p["V2_w"].T, ((0, 0), (0, VPAD - WORD_COUNT)))
    b2 = jnp.pad(p["V2_b"], (0, VPAD - WORD_COUNT),
                 constant_values=-1e30).reshape(1, VPAD)

    logits = _tc_logits(hidden, v2t, b2)
    pev = _tc_pev(logits, pg)
    out = _sc_scatter(pev, vals, idx2)
    return jnp.transpose(out, (1, 0, 2)), cov.reshape(())
